# baseline (speedup 1.0000x reference)
import functools

import jax
import jax.numpy as jnp
from jax import lax
from jax.experimental import pallas as pl
from jax.experimental.pallas import tpu as pltpu
from jax.experimental.pallas import tpu_sc as plsc

VOCAB = 1000000
DIM = 64
R, A, M = 64, 4, 128
RA = R * A
NSRC = RA * M

NC, NS = 2, 16
NW = NC * NS
SPW = NSRC // NW // M
APW = RA // NW

PAIRS_PER_STEP = 16
STEPS = RA // PAIRS_PER_STEP
HM = M // 2


IPW = NSRC // NW
NB = 16
NBATCH = IPW // NB
NPIPE = 4
G8 = VOCAB // 8
L = 16


@functools.cache
def _sc_gather_fn(ipw):
    nbatch = ipw // NB
    mesh = plsc.VectorSubcoreMesh(core_axis_name="c", subcore_axis_name="s")

    @functools.partial(
        pl.kernel,
        mesh=mesh,
        out_type=[
            jax.ShapeDtypeStruct((NW * nbatch, NB, DIM), jnp.float32),
            jax.ShapeDtypeStruct((NW, APW, DIM), jnp.float32),
        ],
        scratch_types=[
            pltpu.VMEM((ipw,), jnp.int32),
            pltpu.VMEM((L,), jnp.int32),
            [pltpu.VMEM((NB, 8, DIM), jnp.float32) for _ in range(NPIPE)],
            [pltpu.VMEM((NB, DIM), jnp.float32) for _ in range(NPIPE)],
            pltpu.VMEM((L, 8, DIM), jnp.float32),
            pltpu.VMEM((L, DIM), jnp.float32),
            [pltpu.SemaphoreType.DMA for _ in range(NPIPE)],
            pltpu.SemaphoreType.DMA,
        ],
        compiler_params=pltpu.CompilerParams(needs_layout_passes=False),
    )
    def _sc_gather(table_hbm, sidx_hbm, aidx_hbm, src_out, aux_out,
                   sidx_v, aidx_v, bufs, ebufs, abuf, aebuf, sems, osem):
        wid = lax.axis_index("s") * NC + lax.axis_index("c")
        pltpu.sync_copy(sidx_hbm.at[wid], sidx_v)
        pltpu.sync_copy(aidx_hbm.at[pl.ds(wid * APW, APW)],
                        aidx_v.at[pl.ds(0, APW)])

        lanes = lax.iota(jnp.int32, L)

        def lane_scalar(vec, b):
            return jnp.max(jnp.where(lanes == b, vec, -1))

        av = jnp.where(lanes < APW, aidx_v[pl.ds(0, L)], 0)
        agvec = lax.shift_right_logical(av, 3)
        aovec = lax.bitwise_and(av, 7)
        for b in range(APW):
            pltpu.async_copy(table_hbm.at[lane_scalar(agvec, b)],
                             abuf.at[b], osem)
        for b in range(APW):
            pltpu.make_async_copy(table_hbm.at[0], abuf.at[b], osem).wait()
        for col in range(DIM):
            cvec = jnp.full((L,), col, jnp.int32)
            x = plsc.load_gather(abuf, [lanes, aovec, cvec])
            plsc.store_scatter(aebuf, [lanes, cvec], x)
        pltpu.sync_copy(aebuf.at[pl.ds(0, APW)], aux_out.at[wid])

        def fire(batch, buf, sem):
            svec = sidx_v[pl.ds(batch * NB, NB)]
            gvec = lax.shift_right_logical(svec, 3)
            for b in range(NB):
                pltpu.async_copy(table_hbm.at[lane_scalar(gvec, b)],
                                 buf.at[b], sem)

        def drain_extract(batch, buf, ebuf, sem):
            for b in range(NB):
                pltpu.make_async_copy(table_hbm.at[0], buf.at[b], sem).wait()
            svec = sidx_v[pl.ds(batch * NB, NB)]
            ovec = lax.bitwise_and(svec, 7)
            for col in range(DIM):
                cvec = jnp.full((L,), col, jnp.int32)
                x = plsc.load_gather(buf, [lanes, ovec, cvec])
                plsc.store_scatter(ebuf, [lanes, cvec], x)
            pltpu.async_copy(ebuf, src_out.at[wid * nbatch + batch], osem)

        for j in range(NPIPE):
            fire(j, bufs[j], sems[j])

        def step(u, carry):
            for j in range(NPIPE):
                b = u * NPIPE + j

                @pl.when(u > 0)
                def _wait_out():
                    pltpu.make_async_copy(
                        src_out.at[0], ebufs[j], osem).wait()

                drain_extract(b, bufs[j], ebufs[j], sems[j])

                @pl.when(u < nbatch // NPIPE - 1)
                def _refire():
                    fire(b + NPIPE, bufs[j], sems[j])
            return carry
        lax.fori_loop(0, nbatch // NPIPE, step, 0)

        for j in range(NPIPE):
            pltpu.make_async_copy(src_out.at[0], ebufs[j], osem).wait()

    return _sc_gather


def _tc_body(src_ref, aux_ref, carry_ref, out_ref, acc128, acc64, smax_ref,
             *, nsteps, final):
    i = pl.program_id(0)

    @pl.when(i == 0)
    def _init():
        acc128[...] = jnp.zeros((1, M), jnp.float32)
        acc64[...] = jnp.zeros((1, DIM), jnp.float32)
        smax_ref[0] = carry_ref[1]

    aux_blk = aux_ref[...]

    smax = jnp.float32(0.0)
    grow, trow = None, None
    for rr in range(PAIRS_PER_STEP // A):
        prod = None
        for aa in range(A):
            p = rr * A + aa
            Xp = src_ref[pl.ds(p * HM, HM), :]
            X = jnp.concatenate([Xp[:, :DIM], Xp[:, DIM:]], axis=0)
            xn = X / (jnp.sqrt(
                jnp.sum(X * X, axis=1, keepdims=True)) + 1e-12)
            G = lax.dot_general(
                xn, xn, (((1,), (1,)), ((), ())),
                preferred_element_type=jnp.float32)
            gshort = jnp.sum(jnp.abs(G), axis=0, keepdims=True)
            tshort = jnp.sum(xn * xn, axis=0, keepdims=True)
            grow = gshort if grow is None else grow + gshort
            trow = tshort if trow is None else trow + tshort
            ar = aux_blk[p // 2:p // 2 + 1,
                         (p % 2) * DIM:(p % 2 + 1) * DIM]
            an = ar / (jnp.sqrt(
                jnp.sum(ar * ar, axis=1, keepdims=True)) + 1e-12)
            s = jnp.abs(lax.dot_general(
                xn, an, (((1,), (1,)), ((), ())),
                preferred_element_type=jnp.float32))
            prod = s if prod is None else prod * s
        smax = jnp.maximum(smax, jnp.max(prod))

    acc128[...] += grow
    acc64[...] += trow
    smax_ref[0] = jnp.maximum(smax_ref[0], smax)

    @pl.when(i == nsteps - 1)
    def _fin():
        sloss = 0.5 * (jnp.sum(acc128[...]) - jnp.sum(acc64[...])) \
            + carry_ref[0]
        if final:
            om = 1.0 - smax_ref[0]
            om2 = om * om
            om4 = om2 * om2
            om8 = om4 * om4
            out_ref[0] = om8 * om2 + sloss
            out_ref[1] = om
        else:
            out_ref[0] = sloss
            out_ref[1] = smax_ref[0]


def _tc_call(src, aux, carry, final):
    npairs = src.shape[0] // HM
    nsteps = npairs // PAIRS_PER_STEP
    body = functools.partial(_tc_body, nsteps=nsteps, final=final)
    return pl.pallas_call(
        body,
        grid=(nsteps,),
        in_specs=[
            pl.BlockSpec((PAIRS_PER_STEP * HM, 2 * DIM), lambda i: (i, 0)),
            pl.BlockSpec((PAIRS_PER_STEP // 2, 2 * DIM), lambda i: (i, 0)),
            pl.BlockSpec(memory_space=pltpu.SMEM),
        ],
        out_specs=pl.BlockSpec(memory_space=pltpu.SMEM),
        out_shape=jax.ShapeDtypeStruct((2,), jnp.float32),
        scratch_shapes=[
            pltpu.VMEM((1, M), jnp.float32),
            pltpu.VMEM((1, DIM), jnp.float32),
            pltpu.SMEM((1,), jnp.float32),
        ],
    )(src, aux, carry)


def kernel(table, Temp, aux_idx, source_idx):
    del Temp
    table8 = table.reshape(G8, 8, DIM)
    sidx0 = lax.slice(source_idx.reshape(NSRC), (0,), (NSRC // 2,))
    sidx1 = lax.slice(source_idx.reshape(NSRC), (NSRC // 2,), (NSRC,))
    aidx = aux_idx.reshape(RA)
    gather = _sc_gather_fn(NSRC // 2 // NW)
    src0_rows, aux_rows = gather(table8, sidx0.reshape(NW, -1), aidx)
    src1_rows, _ = gather(table8, sidx1.reshape(NW, -1), aidx)
    src0 = src0_rows.reshape(NSRC // 4, 2 * DIM)
    src1 = src1_rows.reshape(NSRC // 4, 2 * DIM)
    aux = aux_rows.reshape(RA // 2, 2 * DIM)
    aux0 = lax.slice(aux, (0, 0), (RA // 4, 2 * DIM))
    aux1 = lax.slice(aux, (RA // 4, 0), (RA // 2, 2 * DIM))

    carry = _tc_call(src0, aux0, jnp.zeros((2,), jnp.float32), final=False)
    out = _tc_call(src1, aux1, carry, final=True)
    return out


# speedup vs baseline: 1.0276x; 1.0276x over previous
import functools

import jax
import jax.numpy as jnp
from jax import lax
from jax.experimental import pallas as pl
from jax.experimental.pallas import tpu as pltpu
from jax.experimental.pallas import tpu_sc as plsc

VOCAB = 1000000
DIM = 64
R, A, M = 64, 4, 128
RA = R * A
NSRC = RA * M

NC, NS = 2, 16
NW = NC * NS
SPW = NSRC // NW // M
APW = RA // NW

PAIRS_PER_STEP = 16
STEPS = RA // PAIRS_PER_STEP
HM = M // 2


IPW = NSRC // NW
NB = 16
NBATCH = IPW // NB
NPIPE = 4
G8 = VOCAB // 8
L = 16


@functools.cache
def _sc_gather_fn():
    mesh = plsc.VectorSubcoreMesh(core_axis_name="c", subcore_axis_name="s")

    @functools.partial(
        pl.kernel,
        mesh=mesh,
        out_type=[
            jax.ShapeDtypeStruct((NW * NBATCH, NB, DIM), jnp.float32),
            jax.ShapeDtypeStruct((NW, APW, DIM), jnp.float32),
        ],
        scratch_types=[
            pltpu.VMEM((IPW,), jnp.int32),
            pltpu.VMEM((L,), jnp.int32),
            [pltpu.VMEM((NB, 8, DIM), jnp.float32) for _ in range(NPIPE)],
            [pltpu.VMEM((NB, DIM), jnp.float32) for _ in range(NPIPE)],
            pltpu.VMEM((L, 8, DIM), jnp.float32),
            pltpu.VMEM((L, DIM), jnp.float32),
            [pltpu.SemaphoreType.DMA for _ in range(NPIPE)],
            pltpu.SemaphoreType.DMA,
        ],
        compiler_params=pltpu.CompilerParams(needs_layout_passes=False),
    )
    def _sc_gather(table_hbm, sidx_hbm, aidx_hbm, src_out, aux_out,
                   sidx_v, aidx_v, bufs, ebufs, abuf, aebuf, sems, osem):
        wid = lax.axis_index("s") * NC + lax.axis_index("c")
        pltpu.sync_copy(sidx_hbm.at[wid], sidx_v)
        pltpu.sync_copy(aidx_hbm.at[pl.ds(wid * APW, APW)],
                        aidx_v.at[pl.ds(0, APW)])

        lanes = lax.iota(jnp.int32, L)

        def lane_scalar(vec, b):
            return jnp.max(jnp.where(lanes == b, vec, -1))

        av = jnp.where(lanes < APW, aidx_v[pl.ds(0, L)], 0)
        agvec = lax.shift_right_logical(av, 3)
        aovec = lax.bitwise_and(av, 7)
        for b in range(APW):
            pltpu.async_copy(table_hbm.at[lane_scalar(agvec, b)],
                             abuf.at[b], osem)
        for b in range(APW):
            pltpu.make_async_copy(table_hbm.at[0], abuf.at[b], osem).wait()
        for col in range(DIM):
            cvec = jnp.full((L,), col, jnp.int32)
            x = plsc.load_gather(abuf, [lanes, aovec, cvec])
            plsc.store_scatter(aebuf, [lanes, cvec], x)
        pltpu.sync_copy(aebuf.at[pl.ds(0, APW)], aux_out.at[wid])

        def fire(batch, buf, sem):
            svec = sidx_v[pl.ds(batch * NB, NB)]
            gvec = lax.shift_right_logical(svec, 3)
            for b in range(NB):
                pltpu.async_copy(table_hbm.at[lane_scalar(gvec, b)],
                                 buf.at[b], sem)

        def drain_extract(batch, buf, ebuf, sem):
            for b in range(NB):
                pltpu.make_async_copy(table_hbm.at[0], buf.at[b], sem).wait()
            svec = sidx_v[pl.ds(batch * NB, NB)]
            ovec = lax.bitwise_and(svec, 7)
            for col in range(DIM):
                cvec = jnp.full((L,), col, jnp.int32)
                x = plsc.load_gather(buf, [lanes, ovec, cvec])
                plsc.store_scatter(ebuf, [lanes, cvec], x)
            pltpu.async_copy(ebuf, src_out.at[wid * NBATCH + batch], osem)

        for j in range(NPIPE):
            fire(j, bufs[j], sems[j])

        def step(u, carry):
            for j in range(NPIPE):
                b = u * NPIPE + j

                @pl.when(u > 0)
                def _wait_out():
                    pltpu.make_async_copy(
                        src_out.at[0], ebufs[j], osem).wait()

                drain_extract(b, bufs[j], ebufs[j], sems[j])

                @pl.when(u < NBATCH // NPIPE - 1)
                def _refire():
                    fire(b + NPIPE, bufs[j], sems[j])
            return carry
        lax.fori_loop(0, NBATCH // NPIPE, step, 0)

        for j in range(NPIPE):
            pltpu.make_async_copy(src_out.at[0], ebufs[j], osem).wait()

    return _sc_gather


def _tc_body(src_ref, aux_ref, out_ref, acc128, acc64, smax_ref):
    i = pl.program_id(0)

    @pl.when(i == 0)
    def _init():
        acc128[...] = jnp.zeros((1, M), jnp.float32)
        acc64[...] = jnp.zeros((1, DIM), jnp.float32)
        smax_ref[0] = 0.0

    aux_blk = aux_ref[...]

    smax = jnp.float32(0.0)
    grow, trow = None, None
    for rr in range(PAIRS_PER_STEP // A):
        prod = None
        for aa in range(A):
            p = rr * A + aa
            Xp = src_ref[pl.ds(p * HM, HM), :]
            X = jnp.concatenate([Xp[:, :DIM], Xp[:, DIM:]], axis=0)
            xn = X / (jnp.sqrt(
                jnp.sum(X * X, axis=1, keepdims=True)) + 1e-12)
            G = lax.dot_general(
                xn, xn, (((1,), (1,)), ((), ())),
                preferred_element_type=jnp.float32)
            gshort = jnp.sum(jnp.abs(G), axis=0, keepdims=True)
            tshort = jnp.sum(xn * xn, axis=0, keepdims=True)
            grow = gshort if grow is None else grow + gshort
            trow = tshort if trow is None else trow + tshort
            ar = aux_blk[p // 2:p // 2 + 1,
                         (p % 2) * DIM:(p % 2 + 1) * DIM]
            an = ar / (jnp.sqrt(
                jnp.sum(ar * ar, axis=1, keepdims=True)) + 1e-12)
            s = jnp.abs(lax.dot_general(
                xn, an, (((1,), (1,)), ((), ())),
                preferred_element_type=jnp.float32))
            prod = s if prod is None else prod * s
        smax = jnp.maximum(smax, jnp.max(prod))

    acc128[...] += grow
    acc64[...] += trow
    smax_ref[0] = jnp.maximum(smax_ref[0], smax)

    @pl.when(i == STEPS - 1)
    def _fin():
        sloss = 0.5 * (jnp.sum(acc128[...]) - jnp.sum(acc64[...]))
        om = 1.0 - smax_ref[0]
        om2 = om * om
        om4 = om2 * om2
        om8 = om4 * om4
        out_ref[0] = om8 * om2 + sloss
        out_ref[1] = om


def kernel(table, Temp, aux_idx, source_idx):
    del Temp
    table8 = table.reshape(G8, 8, DIM)
    sidx = source_idx.reshape(NW, IPW)
    aidx = aux_idx.reshape(RA)
    src_rows, aux_rows = _sc_gather_fn()(table8, sidx, aidx)
    src = src_rows.reshape(NSRC // 2, 2 * DIM)
    aux = aux_rows.reshape(RA // 2, 2 * DIM)

    out = pl.pallas_call(
        _tc_body,
        grid=(STEPS,),
        in_specs=[
            pl.BlockSpec((PAIRS_PER_STEP * HM, 2 * DIM), lambda i: (i, 0)),
            pl.BlockSpec((PAIRS_PER_STEP // 2, 2 * DIM), lambda i: (i, 0)),
        ],
        out_specs=pl.BlockSpec(memory_space=pltpu.SMEM),
        out_shape=jax.ShapeDtypeStruct((2,), jnp.float32),
        scratch_shapes=[
            pltpu.VMEM((1, M), jnp.float32),
            pltpu.VMEM((1, DIM), jnp.float32),
            pltpu.SMEM((1,), jnp.float32),
        ],
    )(src, aux)
    return out
